# Initial kernel scaffold; baseline (speedup 1.0000x reference)
import jax
import jax.numpy as jnp
from jax.experimental import pallas as pl


def kernel(x, source, batch_source, target, batch_target, W1, b1, g1, bt1, W2, b2, g2, bt2, W3, b3, g3, bt3, Cw1, Cb1, g4, bt4, Cw2, Cb2, g5, bt5, Cw3, Cb3, Wl, bl):
    raise NotImplementedError("write your pallas kernel here")


# speedup vs baseline: 4.1203x; 4.1203x over previous
import functools

import jax
import jax.numpy as jnp
from jax import lax
from jax.experimental import pallas as pl
from jax.experimental.pallas import tpu as pltpu
from jax.experimental.pallas import tpu_sc as plsc

N = 8192
D = 3
K = 16
C_IN = 128
C_DELTA = 32
C_MID = C_IN + C_DELTA
C_OUT = 256
EPS = 1e-5

KNN_BLK = 256
BLK = 128
NB = N // BLK


def _elu(v):
    return jnp.where(v > 0, v, jnp.exp(v) - 1.0)


def _knn_body(t_ref, s_ref, idx_ref):
    t = t_ref[...]
    s = s_ref[...]
    tn2 = ((t[:, 0] * t[:, 0] + t[:, 1] * t[:, 1]) + t[:, 2] * t[:, 2])[:, None]
    sn2 = ((s[:, 0] * s[:, 0] + s[:, 1] * s[:, 1]) + s[:, 2] * s[:, 2])[None, :]
    m = jnp.dot(t, s.T, preferred_element_type=jnp.float32)
    d = (tn2 + sn2) - 2.0 * m
    iota = lax.broadcasted_iota(jnp.int32, d.shape, 1)
    big = jnp.int32(2**30)
    for k in range(K):
        v = jnp.min(d, axis=1, keepdims=True)
        cand = jnp.where(d == v, iota, big)
        j = jnp.min(cand, axis=1, keepdims=True)
        idx_ref[:, k : k + 1] = j
        d = jnp.where(cand == j, jnp.inf, d)


def _knn(tpad, spad):
    return pl.pallas_call(
        _knn_body,
        grid=(N // KNN_BLK,),
        in_specs=[
            pl.BlockSpec((KNN_BLK, 8), lambda i: (i, 0)),
            pl.BlockSpec((N, 8), lambda i: (0, 0)),
        ],
        out_specs=pl.BlockSpec((KNN_BLK, K), lambda i: (i, 0)),
        out_shape=jax.ShapeDtypeStruct((N, K), jnp.int32),
    )(tpad, spad)


_NW = 32
_B = N * K
_BPW = _B // _NW
_XCH = 512


def _sc_gather(x, spad128, col):
    mesh = plsc.VectorSubcoreMesh(core_axis_name="c", subcore_axis_name="s")

    @functools.partial(
        pl.kernel,
        out_type=(
            jax.ShapeDtypeStruct((_B, C_IN), jnp.float32),
            jax.ShapeDtypeStruct((_B, C_IN), jnp.float32),
        ),
        mesh=mesh,
        scratch_types=[
            pltpu.VMEM((_BPW,), jnp.int32),
            pltpu.VMEM((_XCH, C_IN), jnp.float32),
            pltpu.SemaphoreType.DMA,
        ],
    )
    def k(x_hbm, s_hbm, col_hbm, xg_hbm, sg_hbm, idx_v, buf, sem):
        wid = lax.axis_index("s") * 2 + lax.axis_index("c")
        base = wid * _BPW
        pltpu.sync_copy(col_hbm.at[pl.ds(base, _BPW)], idx_v)
        for c in range(_BPW // _XCH):
            ids = idx_v.at[pl.ds(c * _XCH, _XCH)]
            pltpu.async_copy(s_hbm.at[ids], buf, sem).wait()
            pltpu.sync_copy(buf, sg_hbm.at[pl.ds(base + c * _XCH, _XCH)])
            pltpu.async_copy(x_hbm.at[ids], buf, sem).wait()
            pltpu.sync_copy(buf, xg_hbm.at[pl.ds(base + c * _XCH, _XCH)])

    return k(x, spad128, col)


def _pass_a_body(sg_ref, t_ref, w1_ref, b1_ref, w3_ref, b3_ref,
                 e1_ref, e3_ref, st1_ref, st3_ref):
    pos = sg_ref[...] - t_ref[...][:, None, :]
    pos2 = pos.reshape(BLK * K, 16)
    a1 = jnp.dot(pos2, w1_ref[...], preferred_element_type=jnp.float32, precision=jax.lax.Precision.HIGHEST) + b1_ref[...]
    e1 = _elu(a1)
    e1_ref[...] = e1.reshape(BLK, K, C_DELTA)
    st1_ref[...] = jnp.concatenate(
        [jnp.sum(e1, axis=0), jnp.sum(e1 * e1, axis=0)]).reshape(1, 1, 2 * C_DELTA)
    p = pos.reshape(BLK, K * 16)
    a3 = jnp.dot(p, w3_ref[...], preferred_element_type=jnp.float32, precision=jax.lax.Precision.HIGHEST) + b3_ref[...]
    e3 = _elu(a3)
    e3_ref[...] = e3
    st3_ref[...] = jnp.concatenate(
        [jnp.sum(e3, axis=0), jnp.sum(e3 * e3, axis=0)]).reshape(1, 1, 512)


def _pass_a(srcg, tpad16, w1p, b1r, w3p, b3r):
    return pl.pallas_call(
        _pass_a_body,
        grid=(NB,),
        in_specs=[
            pl.BlockSpec((BLK, K, 16), lambda i: (i, 0, 0)),
            pl.BlockSpec((BLK, 16), lambda i: (i, 0)),
            pl.BlockSpec((16, C_DELTA), lambda i: (0, 0)),
            pl.BlockSpec((1, C_DELTA), lambda i: (0, 0)),
            pl.BlockSpec((256, 256), lambda i: (0, 0)),
            pl.BlockSpec((1, 256), lambda i: (0, 0)),
        ],
        out_specs=[
            pl.BlockSpec((BLK, K, C_DELTA), lambda i: (i, 0, 0)),
            pl.BlockSpec((BLK, 256), lambda i: (i, 0)),
            pl.BlockSpec((1, 1, 2 * C_DELTA), lambda i: (i, 0, 0)),
            pl.BlockSpec((1, 1, 512), lambda i: (i, 0, 0)),
        ],
        out_shape=[
            jax.ShapeDtypeStruct((N, K, C_DELTA), jnp.float32),
            jax.ShapeDtypeStruct((N, 256), jnp.float32),
            jax.ShapeDtypeStruct((NB, 1, 2 * C_DELTA), jnp.float32),
            jax.ShapeDtypeStruct((NB, 1, 512), jnp.float32),
        ],
    )(srcg, tpad16, w1p, b1r, w3p, b3r)


def _pass_b_body(e1_ref, sc1_ref, sh1_ref, w2_ref, b2_ref,
                 e3_ref, sc3_ref, sh3_ref, bd1_ref, cb1_ref,
                 e2_ref, e4_ref, st2_ref, st4_ref):
    h1 = e1_ref[...] * sc1_ref[...][None] + sh1_ref[...][None]
    a2 = jnp.dot(h1.reshape(BLK * K, C_DELTA), w2_ref[...],
                 preferred_element_type=jnp.float32, precision=jax.lax.Precision.HIGHEST) + b2_ref[...]
    e2 = _elu(a2)
    e2_ref[...] = e2.reshape(BLK, K, C_DELTA)
    st2_ref[...] = jnp.concatenate(
        [jnp.sum(e2, axis=0), jnp.sum(e2 * e2, axis=0)]).reshape(1, 1, 2 * C_DELTA)
    t1 = e3_ref[...] * sc3_ref[...] + sh3_ref[...]
    t2 = jnp.dot(t1, bd1_ref[...], preferred_element_type=jnp.float32, precision=jax.lax.Precision.HIGHEST) + cb1_ref[...]
    e4 = _elu(t2)
    e4_ref[...] = e4
    st4_ref[...] = jnp.concatenate(
        [jnp.sum(e4, axis=0), jnp.sum(e4 * e4, axis=0)]).reshape(1, 1, 512)


def _pass_b(e1, sc1, sh1, w2t, b2r, e3, sc3, sh3, bd1, cb1r):
    return pl.pallas_call(
        _pass_b_body,
        grid=(NB,),
        in_specs=[
            pl.BlockSpec((BLK, K, C_DELTA), lambda i: (i, 0, 0)),
            pl.BlockSpec((1, C_DELTA), lambda i: (0, 0)),
            pl.BlockSpec((1, C_DELTA), lambda i: (0, 0)),
            pl.BlockSpec((C_DELTA, C_DELTA), lambda i: (0, 0)),
            pl.BlockSpec((1, C_DELTA), lambda i: (0, 0)),
            pl.BlockSpec((BLK, 256), lambda i: (i, 0)),
            pl.BlockSpec((1, 256), lambda i: (0, 0)),
            pl.BlockSpec((1, 256), lambda i: (0, 0)),
            pl.BlockSpec((256, 256), lambda i: (0, 0)),
            pl.BlockSpec((1, 256), lambda i: (0, 0)),
        ],
        out_specs=[
            pl.BlockSpec((BLK, K, C_DELTA), lambda i: (i, 0, 0)),
            pl.BlockSpec((BLK, 256), lambda i: (i, 0)),
            pl.BlockSpec((1, 1, 2 * C_DELTA), lambda i: (i, 0, 0)),
            pl.BlockSpec((1, 1, 512), lambda i: (i, 0, 0)),
        ],
        out_shape=[
            jax.ShapeDtypeStruct((N, K, C_DELTA), jnp.float32),
            jax.ShapeDtypeStruct((N, 256), jnp.float32),
            jax.ShapeDtypeStruct((NB, 1, 2 * C_DELTA), jnp.float32),
            jax.ShapeDtypeStruct((NB, 1, 512), jnp.float32),
        ],
    )(e1, sc1, sh1, w2t, b2r, e3, sc3, sh3, bd1, cb1r)


def _pass_c_body(e4_ref, sc4_ref, sh4_ref, bd2_ref, cb2_ref, t4_ref, st5_ref):
    t3 = e4_ref[...] * sc4_ref[...] + sh4_ref[...]
    t4 = jnp.dot(t3, bd2_ref[...], preferred_element_type=jnp.float32, precision=jax.lax.Precision.HIGHEST) + cb2_ref[...]
    t4_ref[...] = t4
    st5_ref[...] = jnp.concatenate(
        [jnp.sum(t4, axis=0), jnp.sum(t4 * t4, axis=0)]).reshape(1, 1, 512)


def _pass_c(e4, sc4, sh4, bd2, cb2r):
    return pl.pallas_call(
        _pass_c_body,
        grid=(NB,),
        in_specs=[
            pl.BlockSpec((BLK, 256), lambda i: (i, 0)),
            pl.BlockSpec((1, 256), lambda i: (0, 0)),
            pl.BlockSpec((1, 256), lambda i: (0, 0)),
            pl.BlockSpec((256, 256), lambda i: (0, 0)),
            pl.BlockSpec((1, 256), lambda i: (0, 0)),
        ],
        out_specs=[
            pl.BlockSpec((BLK, 256), lambda i: (i, 0)),
            pl.BlockSpec((1, 1, 512), lambda i: (i, 0, 0)),
        ],
        out_shape=[
            jax.ShapeDtypeStruct((N, 256), jnp.float32),
            jax.ShapeDtypeStruct((NB, 1, 512), jnp.float32),
        ],
    )(e4, sc4, sh4, bd2, cb2r)


def _pass_d_body(e2_ref, sc2_ref, sh2_ref, t4_ref, sc5_ref, sh5_ref,
                 xg_ref, c3a_ref, c3b_ref, wa_ref, wb_ref, bl_ref, out_ref):
    bf = jnp.bfloat16
    f32 = jnp.float32
    h2 = e2_ref[...] * sc2_ref[...][None] + sh2_ref[...][None]
    tb = t4_ref[...] * sc5_ref[...] + sh5_ref[...]
    h2 = h2.astype(bf).astype(f32)
    tb = tb.astype(bf).astype(f32)
    xg = xg_ref[...].astype(bf).astype(f32)
    c3a = c3a_ref[...]
    c3b = c3b_ref[...]
    oah = jnp.zeros((BLK, C_DELTA), jnp.float32)
    obh = jnp.zeros((BLK, C_DELTA), jnp.float32)
    oag = jnp.zeros((BLK, C_IN), jnp.float32)
    obg = jnp.zeros((BLK, C_IN), jnp.float32)
    for k in range(K):
        xth = jnp.zeros((BLK, C_DELTA), jnp.float32)
        xtg = jnp.zeros((BLK, C_IN), jnp.float32)
        for j in range(K):
            f = tb[:, 16 * k + j][:, None]
            xth = xth + f * h2[:, j, :]
            xtg = xtg + f * xg[:, j, :]
        xth = xth.astype(bf).astype(f32)
        xtg = xtg.astype(bf).astype(f32)
        oah = oah + xth * c3a[k, :C_DELTA][None, :]
        oag = oag + xtg * c3a[k, C_DELTA:][None, :]
        obh = obh + xth * c3b[k, :C_DELTA][None, :]
        obg = obg + xtg * c3b[k, C_DELTA:][None, :]
    hi = jax.lax.Precision.HIGHEST
    out = (jnp.dot(oah, wa_ref[:C_DELTA, :], preferred_element_type=f32, precision=hi)
           + jnp.dot(oag, wa_ref[C_DELTA:, :], preferred_element_type=f32, precision=hi)
           + jnp.dot(obh, wb_ref[:C_DELTA, :], preferred_element_type=f32, precision=hi)
           + jnp.dot(obg, wb_ref[C_DELTA:, :], preferred_element_type=f32, precision=hi))
    out_ref[...] = out + bl_ref[...]


def _pass_d(e2, sc2, sh2, t4, sc5, sh5, xg, c3a, c3b, wat, wbt, blr):
    return pl.pallas_call(
        _pass_d_body,
        grid=(NB,),
        in_specs=[
            pl.BlockSpec((BLK, K, C_DELTA), lambda i: (i, 0, 0)),
            pl.BlockSpec((1, C_DELTA), lambda i: (0, 0)),
            pl.BlockSpec((1, C_DELTA), lambda i: (0, 0)),
            pl.BlockSpec((BLK, 256), lambda i: (i, 0)),
            pl.BlockSpec((1, 256), lambda i: (0, 0)),
            pl.BlockSpec((1, 256), lambda i: (0, 0)),
            pl.BlockSpec((BLK, K, C_IN), lambda i: (i, 0, 0)),
            pl.BlockSpec((K, C_MID), lambda i: (0, 0)),
            pl.BlockSpec((K, C_MID), lambda i: (0, 0)),
            pl.BlockSpec((C_MID, C_OUT), lambda i: (0, 0)),
            pl.BlockSpec((C_MID, C_OUT), lambda i: (0, 0)),
            pl.BlockSpec((1, C_OUT), lambda i: (0, 0)),
        ],
        out_specs=pl.BlockSpec((BLK, C_OUT), lambda i: (i, 0)),
        out_shape=jax.ShapeDtypeStruct((N, C_OUT), jnp.float32),
    )(e2, sc2, sh2, t4, sc5, sh5, xg, c3a, c3b, wat, wbt, blr)


def _bn_coeffs(st, n, g, b):
    s = jnp.sum(st, axis=0).reshape(-1)
    c = s.shape[0] // 2
    m = s[:c] / n
    v = s[c:] / n - m * m
    scale = g / jnp.sqrt(v + EPS)
    shift = b - m * scale
    return scale.reshape(1, c), shift.reshape(1, c)


def _block_diag(cw):
    blocks = jnp.transpose(cw[:, 0, :].reshape(16, 16, 16), (0, 2, 1))
    return (blocks[:, :, None, :] * jnp.eye(16)[:, None, :, None]).reshape(256, 256)


def kernel(x, source, batch_source, target, batch_target, W1, b1, g1, bt1,
           W2, b2, g2, bt2, W3, b3, g3, bt3, Cw1, Cb1, g4, bt4,
           Cw2, Cb2, g5, bt5, Cw3, Cb3, Wl, bl):
    f32 = jnp.float32
    tpad8 = jnp.pad(target, ((0, 0), (0, 5)))
    spad8 = jnp.pad(source, ((0, 0), (0, 5)))
    tpad16 = jnp.pad(target, ((0, 0), (0, 13)))
    spad128 = jnp.pad(source, ((0, 0), (0, 125)))

    idx = _knn(tpad8, spad8)
    col = idx.reshape(_B)
    xg, srcg = _sc_gather(x, spad128, col)
    srcg = srcg[:, :16].reshape(N, K, 16)
    xg = xg.reshape(N, K, C_IN)

    w1p = jnp.zeros((16, C_DELTA), f32).at[:D, :].set(W1.T)
    w3p = jnp.zeros((K, 16, 256), f32).at[:, :D, :].set(
        W3.T.reshape(K, D, 256)).reshape(256, 256)
    bd1 = _block_diag(Cw1)
    bd2 = _block_diag(Cw2)
    cw3r = Cw3[:, 0, :].reshape(C_MID, 2, K)
    c3a = cw3r[:, 0, :].T
    c3b = cw3r[:, 1, :].T
    wat = Wl[:, 0::2].T
    wbt = Wl[:, 1::2].T
    bl_eff = (bl + Wl @ Cb3).reshape(1, C_OUT)

    e1, e3, st1, st3 = _pass_a(srcg, tpad16, w1p, b1.reshape(1, -1),
                               w3p, b3.reshape(1, -1))
    sc1, sh1 = _bn_coeffs(st1, float(_B), g1, bt1)
    sc3, sh3 = _bn_coeffs(st3, float(N), g3, bt3)
    e2, e4, st2, st4 = _pass_b(e1, sc1, sh1, W2.T, b2.reshape(1, -1),
                               e3, sc3, sh3, bd1, Cb1.reshape(1, -1))
    sc2, sh2 = _bn_coeffs(st2, float(_B), g2, bt2)
    sc4, sh4 = _bn_coeffs(st4, float(N), g4, bt4)
    t4, st5 = _pass_c(e4, sc4, sh4, bd2, Cb2.reshape(1, -1))
    sc5, sh5 = _bn_coeffs(st5, float(N), g5, bt5)
    return _pass_d(e2, sc2, sh2, t4, sc5, sh5, xg, c3a, c3b, wat, wbt, bl_eff)
